# baseline (speedup 1.0000x reference)
import jax
import jax.numpy as jnp
from jax.experimental import pallas as pl

_S = 8


def _pool_conv_broadcast_kernel(*refs):
    xs = refs[:_S]
    wt_ref, b_ref, o_ref = refs[_S], refs[_S + 1], refs[_S + 2]
    hw = o_ref.shape[1]
    ones = jnp.ones((1, xs[0].shape[2]), jnp.float32)
    m = jnp.dot(ones, xs[0][0, 0], preferred_element_type=jnp.float32)
    for x in xs[1:]:
        m = m + jnp.dot(ones, x[0, 0],
                        preferred_element_type=jnp.float32)
    pooled = jnp.dot(m * (1.0 / hw), wt_ref[...],
                     preferred_element_type=jnp.float32) + b_ref[...]
    o_ref[0] = jnp.broadcast_to(pooled, o_ref.shape[1:])


def kernel(features, depth, W, b):
    del depth
    N, C, H, Wd = features.shape
    O = W.shape[0]
    HW = H * Wd
    R = HW // _S
    x = features.transpose(0, 2, 3, 1).reshape(N, _S, R, C)
    wt = W.reshape(O, C).T
    b2 = b.reshape(1, O)
    x_specs = [
        pl.BlockSpec((1, 1, R, C), lambda i, s=s: (i, s, 0, 0)) for s in range(_S)
    ]
    out = pl.pallas_call(
        _pool_conv_broadcast_kernel,
        grid=(N,),
        in_specs=x_specs + [
            pl.BlockSpec((C, O), lambda i: (0, 0)),
            pl.BlockSpec((1, O), lambda i: (0, 0)),
        ],
        out_specs=pl.BlockSpec((1, HW, O), lambda i: (i, 0, 0)),
        out_shape=jax.ShapeDtypeStruct((N, HW, O), jnp.float32),
    )(*([x] * _S), wt, b2)
    return out.reshape(N, H, Wd, O).transpose(0, 3, 1, 2)


# speedup vs baseline: 1.0375x; 1.0375x over previous
import jax
import jax.numpy as jnp
from jax.experimental import pallas as pl
from jax.experimental.pallas import tpu as pltpu

_S = 4


def _pool_conv_broadcast_kernel(nb, *refs):
    xs = refs[:_S]
    wt_ref, b_ref, o_ref, m_ref = refs[_S], refs[_S + 1], refs[_S + 2], refs[_S + 3]
    i = pl.program_id(0)
    hw = o_ref.shape[1]

    @pl.when(i > 0)
    def _emit_prev():
        pooled = jnp.dot(m_ref[...] * (1.0 / hw), wt_ref[...],
                         preferred_element_type=jnp.float32) + b_ref[...]
        o_ref[0] = jnp.broadcast_to(pooled, o_ref.shape[1:])

    @pl.when(i < nb)
    def _reduce_cur():
        m = xs[0][0, 0].sum(axis=0, keepdims=True)
        for x in xs[1:]:
            m = m + x[0, 0].sum(axis=0, keepdims=True)
        m_ref[...] = m


def kernel(features, depth, W, b):
    del depth
    N, C, H, Wd = features.shape
    O = W.shape[0]
    HW = H * Wd
    R = HW // _S
    x = features.transpose(0, 2, 3, 1).reshape(N, _S, R, C)
    wt = W.reshape(O, C).T
    b2 = b.reshape(1, O)
    x_specs = [
        pl.BlockSpec((1, 1, R, C),
                     lambda i, s=s: (jnp.minimum(i, N - 1), s, 0, 0))
        for s in range(_S)
    ]
    import functools
    out = pl.pallas_call(
        functools.partial(_pool_conv_broadcast_kernel, N),
        grid=(N + 1,),
        in_specs=x_specs + [
            pl.BlockSpec((C, O), lambda i: (0, 0)),
            pl.BlockSpec((1, O), lambda i: (0, 0)),
        ],
        out_specs=pl.BlockSpec((1, HW, O),
                               lambda i: (jnp.maximum(i - 1, 0), 0, 0)),
        out_shape=jax.ShapeDtypeStruct((N, HW, O), jnp.float32),
        scratch_shapes=[pltpu.VMEM((1, C), jnp.float32)],
    )(*([x] * _S), wt, b2)
    return out.reshape(N, H, Wd, O).transpose(0, 3, 1, 2)


# speedup vs baseline: 1.0403x; 1.0028x over previous
import jax
import jax.numpy as jnp
from jax.experimental import pallas as pl

_S = 4


def _pool_conv_broadcast_kernel(*refs):
    xs = refs[:_S]
    wt_ref, b_ref, o_ref = refs[_S], refs[_S + 1], refs[_S + 2]
    hw = o_ref.shape[1]
    m = xs[0][0, 0].sum(axis=0, keepdims=True)
    for x in xs[1:]:
        m = m + x[0, 0].sum(axis=0, keepdims=True)
    pooled = jnp.dot(m * (1.0 / hw), wt_ref[...],
                     preferred_element_type=jnp.float32) + b_ref[...]
    o_ref[0] = jnp.broadcast_to(pooled, o_ref.shape[1:])


def kernel(features, depth, W, b):
    del depth
    N, C, H, Wd = features.shape
    O = W.shape[0]
    HW = H * Wd
    R = HW // _S
    x = features.transpose(0, 2, 3, 1).reshape(N, _S, R, C)
    wt = W.reshape(O, C).T
    b2 = b.reshape(1, O)
    x_specs = [
        pl.BlockSpec((1, 1, R, C), lambda i, s=s: (i, s, 0, 0)) for s in range(_S)
    ]
    out = pl.pallas_call(
        _pool_conv_broadcast_kernel,
        grid=(N,),
        in_specs=x_specs + [
            pl.BlockSpec((C, O), lambda i: (0, 0)),
            pl.BlockSpec((1, O), lambda i: (0, 0)),
        ],
        out_specs=pl.BlockSpec((1, HW, O), lambda i: (i, 0, 0)),
        out_shape=jax.ShapeDtypeStruct((N, HW, O), jnp.float32),
    )(*([x] * _S), wt, b2)
    return out.reshape(N, H, Wd, O).transpose(0, 3, 1, 2)
